# baseline (speedup 1.0000x reference)
import functools

import jax
import jax.numpy as jnp
from jax.experimental import pallas as pl

N = 10000
D_IN = 128
D_OUT = 128
H = 4
NEG_SLOPE = 0.2


def _mm_body(x_ref, wl_ref, wr_ref, xl_ref, xr_ref):
    x = x_ref[...]
    xl_ref[...] = jnp.dot(x, wl_ref[...], preferred_element_type=jnp.float32)
    xr_ref[...] = jnp.dot(x, wr_ref[...], preferred_element_type=jnp.float32)


@jax.jit
def _proj(x, W_l, W_r):
    nb = 25
    bs = N // nb
    return pl.pallas_call(
        _mm_body,
        grid=(nb,),
        in_specs=[
            pl.BlockSpec((bs, D_IN), lambda i: (i, 0)),
            pl.BlockSpec((D_IN, H * D_OUT), lambda i: (0, 0)),
            pl.BlockSpec((D_IN, H * D_OUT), lambda i: (0, 0)),
        ],
        out_specs=[
            pl.BlockSpec((bs, H * D_OUT), lambda i: (i, 0)),
            pl.BlockSpec((bs, H * D_OUT), lambda i: (i, 0)),
        ],
        out_shape=[
            jax.ShapeDtypeStruct((N, H * D_OUT), jnp.float32),
            jax.ShapeDtypeStruct((N, H * D_OUT), jnp.float32),
        ],
    )(x, W_l, W_r)


def kernel(x, edge_index, W_l, W_r, att, bias):
    num_nodes = x.shape[0]
    loop = jnp.arange(num_nodes, dtype=edge_index.dtype)
    src = jnp.concatenate([edge_index[0], loop])
    dst = jnp.concatenate([edge_index[1], loop])
    x_l, x_r = _proj(x, W_l, W_r)
    x_l = x_l.reshape(num_nodes, H, D_OUT)
    x_r = x_r.reshape(num_nodes, H, D_OUT)
    xj = jnp.take(x_l, src, axis=0)
    xi = jnp.take(x_r, dst, axis=0)
    e = jax.nn.leaky_relu(xi + xj, negative_slope=NEG_SLOPE)
    alpha = jnp.einsum('ehd,hd->eh', e, att)
    amax = jax.ops.segment_max(alpha, dst, num_segments=num_nodes)
    amax = jnp.where(jnp.isfinite(amax), amax, 0.0)
    alpha = jnp.exp(alpha - jnp.take(amax, dst, axis=0))
    denom = jax.ops.segment_sum(alpha, dst, num_segments=num_nodes)
    alpha = alpha / (jnp.take(denom, dst, axis=0) + 1e-16)
    msg = xj * alpha[:, :, None]
    out = jax.ops.segment_sum(msg, dst, num_segments=num_nodes)
    out = out.mean(axis=1)
    out = out + bias
    return out


# speedup vs baseline: 18.9030x; 18.9030x over previous
import jax
import jax.numpy as jnp
from jax import lax
from jax.experimental import pallas as pl
from jax.experimental.pallas import tpu as pltpu
from jax.experimental.pallas import tpu_sc as plsc

N = 10000
E = 320000
D = 128
H = 4
NEG = 0.2

NW = 32
W = 80
ETOT = E + N
NWIN = -(-ETOT // (NW * W))
EPAD = NW * W * NWIN
TPE = EPAD // NW
NPAD = 10112
ZCH = 16
RPT = NPAD // 16
ZTL = RPT % ZCH
DRN = 624
DTL = N - 16 * DRN
BS = 400


def _proj_body(x_ref, wl_ref, wr_ref, xl_ref, xr_ref):
    x = x_ref[...]
    xl_ref[...] = jnp.dot(x, wl_ref[...], preferred_element_type=jnp.float32)
    xr_ref[...] = jnp.dot(x, wr_ref[...], preferred_element_type=jnp.float32)


def _proj(x, W_l, W_r):
    return pl.pallas_call(
        _proj_body,
        grid=(N // BS, H),
        in_specs=[
            pl.BlockSpec((BS, D), lambda i, h: (i, 0)),
            pl.BlockSpec((D, D), lambda i, h: (0, h)),
            pl.BlockSpec((D, D), lambda i, h: (0, h)),
        ],
        out_specs=[
            pl.BlockSpec((BS, D), lambda i, h: (h * (N // BS) + i, 0)),
            pl.BlockSpec((BS, D), lambda i, h: (h * (N // BS) + i, 0)),
        ],
        out_shape=[
            jax.ShapeDtypeStruct((H * N, D), jnp.float32),
            jax.ShapeDtypeStruct((H * N, D), jnp.float32),
        ],
    )(x, W_l, W_r)


def _sc_body(xlf, xrf, srcp, dstp, attb, outp, denp,
             att_v, sidx_v, didx_s, didx_e, didx_g, xi_v, xj_v, den_t,
             zero_v, out_sp, sem1, sem2):
    c = lax.axis_index("c")
    s = lax.axis_index("s")
    wid = s * 2 + c
    tbase = wid * TPE

    pltpu.sync_copy(attb, att_v)

    zvec = jnp.zeros((16,), jnp.float32)
    izvec = jnp.zeros((16,), jnp.int32)
    lane0b = lax.iota(jnp.int32, 16) == 0

    @plsc.parallel_loop(0, ZCH)
    def _zinit(r):
        for k in range(D // 16):
            zero_v[r, pl.ds(k * 16, 16)] = zvec

    didx_e[pl.ds(W, 16)] = izvec

    for h in range(H):
        hoff = jnp.full((16,), h * N, jnp.int32)
        zbase = s * RPT

        def _zs(k, carry):
            pltpu.sync_copy(zero_v, out_sp.at[pl.ds(zbase + k * ZCH, ZCH)])
            return carry

        lax.fori_loop(0, RPT // ZCH, _zs, 0)
        if ZTL:
            ztail = zbase + (RPT // ZCH) * ZCH
            pltpu.sync_copy(zero_v.at[pl.ds(0, ZTL)],
                            out_sp.at[pl.ds(ztail, ZTL)])

        @plsc.parallel_loop(0, NPAD // 16, unroll=4)
        def _zden(i):
            den_t[pl.ds(i * 16, 16)] = zvec

        plsc.subcore_barrier()

        def _win(w, carry):
            base = tbase + w * W
            pltpu.sync_copy(srcp.at[pl.ds(base, W)], sidx_v)
            pltpu.sync_copy(dstp.at[pl.ds(base, W)], didx_s)
            pltpu.sync_copy(dstp.at[pl.ds(base, W)], didx_e.at[pl.ds(0, W)])

            @plsc.parallel_loop(0, W // 16)
            def _off(i):
                dv = didx_s[pl.ds(i * 16, 16)]
                didx_g[pl.ds(i * 16, 16)] = jnp.minimum(dv, N - 1) + hoff
                if h:
                    sidx_v[pl.ds(i * 16, 16)] = sidx_v[pl.ds(i * 16, 16)] + hoff

            d1 = pltpu.async_copy(xlf.at[sidx_v], xj_v, sem1)
            d2 = pltpu.async_copy(xrf.at[didx_g], xi_v, sem2)
            d1.wait()
            d2.wait()

            @plsc.parallel_loop(0, W, unroll=2)
            def _edge(e):
                acc = zvec
                xjc = []
                for k in range(D // 16):
                    xj_c = xj_v[e, pl.ds(k * 16, 16)]
                    xi_c = xi_v[e, pl.ds(k * 16, 16)]
                    z = xi_c + xj_c
                    lr = jnp.maximum(z, z * NEG)
                    acc = acc + lr * att_v[h, pl.ds(k * 16, 16)]
                    xjc.append(xj_c)
                pv = jnp.exp(jnp.full((16,), jnp.sum(acc)))
                for k in range(D // 16):
                    xj_v[e, pl.ds(k * 16, 16)] = xjc[k] * pv
                dchunk = didx_e[pl.ds(e, 16)]
                plsc.addupdate_scatter(den_t, [dchunk], pv, mask=lane0b)

            pltpu.sync_copy(xj_v, out_sp.at[didx_s], add=True)
            return carry

        lax.fori_loop(0, NWIN, _win, 0)
        plsc.subcore_barrier()

        hb = (c * H + h) * N
        obase = hb + s * DRN

        def _drain(k, carry):
            pltpu.sync_copy(out_sp.at[pl.ds(s * DRN + k * 48, 48)],
                            xi_v.at[pl.ds(0, 48)])
            pltpu.sync_copy(xi_v.at[pl.ds(0, 48)],
                            outp.at[pl.ds(obase + k * 48, 48)])
            return carry

        lax.fori_loop(0, DRN // 48, _drain, 0)

        @pl.when(s == 15)
        def _tail():
            pltpu.sync_copy(out_sp.at[pl.ds(16 * DRN, DTL)],
                            xi_v.at[pl.ds(0, DTL)])
            pltpu.sync_copy(xi_v.at[pl.ds(0, DTL)],
                            outp.at[pl.ds(hb + 16 * DRN, DTL)])

        dbase = ((c * H + h) * 16 + s) * NPAD
        pltpu.sync_copy(den_t, denp.at[pl.ds(dbase, NPAD)])

        plsc.subcore_barrier()


def _comb_body(o_ref, d_ref, b_ref, out_ref):
    o = o_ref[...]
    dn = d_ref[...]
    sel = jnp.where(
        lax.broadcasted_iota(jnp.int32, (128, 2 * H), 0) // 16
        == lax.broadcasted_iota(jnp.int32, (128, 2 * H), 1), 1.0, 0.0)
    dsum = jnp.dot(dn, sel, preferred_element_type=jnp.float32)
    o2 = o[0:H] + o[H:2 * H]
    acc = jnp.zeros_like(o2[0])
    for h in range(H):
        den_h = dsum[:, h:h + 1] + dsum[:, H + h:H + h + 1] + 1e-16
        acc = acc + o2[h] / den_h
    out_ref[...] = acc * (1.0 / H) + b_ref[...]


def _combine(outp, denp, bias):
    den_t = denp.reshape(2 * H * 16, NPAD).T
    return pl.pallas_call(
        _comb_body,
        grid=(N // BS,),
        in_specs=[
            pl.BlockSpec((2 * H, BS, D), lambda i: (0, i, 0)),
            pl.BlockSpec((BS, 128), lambda i: (i, 0)),
            pl.BlockSpec((1, D), lambda i: (0, 0)),
        ],
        out_specs=pl.BlockSpec((BS, D), lambda i: (i, 0)),
        out_shape=jax.ShapeDtypeStruct((N, D), jnp.float32),
    )(outp.reshape(2 * H, N, D), den_t, bias.reshape(1, D))


def kernel(x, edge_index, W_l, W_r, att, bias):
    xlf, xrf = _proj(x, W_l, W_r)

    loop = jnp.arange(N, dtype=jnp.int32)
    padi = jnp.arange(EPAD - ETOT, dtype=jnp.int32)
    srcp = jnp.concatenate([edge_index[0], loop, padi % N])
    dstp = jnp.concatenate([edge_index[1], loop, N + padi % (NPAD - N)])

    scfn = pl.kernel(
        _sc_body,
        out_type=[
            jax.ShapeDtypeStruct((2 * H * N, D), jnp.float32),
            jax.ShapeDtypeStruct((2 * H * 16 * NPAD,), jnp.float32),
        ],
        mesh=plsc.VectorSubcoreMesh(core_axis_name="c", subcore_axis_name="s"),
        compiler_params=pltpu.CompilerParams(needs_layout_passes=False),
        scratch_types=[
            pltpu.VMEM((H, D), jnp.float32),
            pltpu.VMEM((W,), jnp.int32),
            pltpu.VMEM((W,), jnp.int32),
            pltpu.VMEM((W + 16,), jnp.int32),
            pltpu.VMEM((W,), jnp.int32),
            pltpu.VMEM((W, D), jnp.float32),
            pltpu.VMEM((W, D), jnp.float32),
            pltpu.VMEM((NPAD,), jnp.float32),
            pltpu.VMEM((ZCH, D), jnp.float32),
            pltpu.VMEM_SHARED((NPAD, D), jnp.float32),
            pltpu.SemaphoreType.DMA,
            pltpu.SemaphoreType.DMA,
        ],
    )
    outp, denp = scfn(xlf, xrf, srcp, dstp, att)
    return _combine(outp, denp, bias)


# speedup vs baseline: 19.0888x; 1.0098x over previous
import jax
import jax.numpy as jnp
from jax import lax
from jax.experimental import pallas as pl
from jax.experimental.pallas import tpu as pltpu
from jax.experimental.pallas import tpu_sc as plsc

N = 10000
E = 320000
D = 128
H = 4
NEG = 0.2

NW = 32
W = 80
ETOT = E + N
NWIN = -(-ETOT // (NW * W))
EPAD = NW * W * NWIN
TPE = EPAD // NW
NPAD = 10112
ZCH = 16
RPT = NPAD // 16
ZTL = RPT % ZCH
DRN = 624
DTL = N - 16 * DRN
BS = 400


def _proj_body(x_ref, wl_ref, wr_ref, xl_ref, xr_ref):
    x = x_ref[...]
    xl_ref[...] = jnp.dot(x, wl_ref[...], preferred_element_type=jnp.float32)
    xr_ref[...] = jnp.dot(x, wr_ref[...], preferred_element_type=jnp.float32)


def _proj(x, W_l, W_r):
    return pl.pallas_call(
        _proj_body,
        grid=(N // BS, H),
        in_specs=[
            pl.BlockSpec((BS, D), lambda i, h: (i, 0)),
            pl.BlockSpec((D, D), lambda i, h: (0, h)),
            pl.BlockSpec((D, D), lambda i, h: (0, h)),
        ],
        out_specs=[
            pl.BlockSpec((BS, D), lambda i, h: (h * (N // BS) + i, 0)),
            pl.BlockSpec((BS, D), lambda i, h: (h * (N // BS) + i, 0)),
        ],
        out_shape=[
            jax.ShapeDtypeStruct((H * N, D), jnp.float32),
            jax.ShapeDtypeStruct((H * N, D), jnp.float32),
        ],
    )(x, W_l, W_r)


def _sc_body(xlf, xrf, srcp, dstp, attb, outp, denp,
             att_v, sidx_v, didx_s, didx_e, didx_g, xi_v, xj_v, den_t,
             zero_v, out_sp, sem1, sem2):
    c = lax.axis_index("c")
    s = lax.axis_index("s")
    wid = s * 2 + c
    tbase = wid * TPE

    pltpu.sync_copy(attb, att_v)

    zvec = jnp.zeros((16,), jnp.float32)
    izvec = jnp.zeros((16,), jnp.int32)
    lane0b = lax.iota(jnp.int32, 16) == 0

    @plsc.parallel_loop(0, ZCH)
    def _zinit(r):
        for k in range(D // 16):
            zero_v[r, pl.ds(k * 16, 16)] = zvec

    didx_e[pl.ds(W, 16)] = izvec

    for h in range(H):
        hoff = jnp.full((16,), h * N, jnp.int32)
        zbase = s * RPT

        def _zs(k, carry):
            pltpu.sync_copy(zero_v, out_sp.at[pl.ds(zbase + k * ZCH, ZCH)])
            return carry

        lax.fori_loop(0, RPT // ZCH, _zs, 0)
        if ZTL:
            ztail = zbase + (RPT // ZCH) * ZCH
            pltpu.sync_copy(zero_v.at[pl.ds(0, ZTL)],
                            out_sp.at[pl.ds(ztail, ZTL)])

        @plsc.parallel_loop(0, NPAD // 16, unroll=4)
        def _zden(i):
            den_t[pl.ds(i * 16, 16)] = zvec

        plsc.subcore_barrier()

        def _win(w, carry):
            base = tbase + w * W
            pltpu.sync_copy(srcp.at[pl.ds(base, W)], sidx_v)
            pltpu.sync_copy(dstp.at[pl.ds(base, W)], didx_s)
            pltpu.sync_copy(dstp.at[pl.ds(base, W)], didx_e.at[pl.ds(0, W)])

            @plsc.parallel_loop(0, W // 16)
            def _off(i):
                dv = didx_s[pl.ds(i * 16, 16)]
                didx_g[pl.ds(i * 16, 16)] = jnp.minimum(dv, N - 1) + hoff
                if h:
                    sidx_v[pl.ds(i * 16, 16)] = sidx_v[pl.ds(i * 16, 16)] + hoff

            d1 = pltpu.async_copy(xlf.at[sidx_v], xj_v, sem1)
            d2 = pltpu.async_copy(xrf.at[didx_g], xi_v, sem2)
            d1.wait()
            d2.wait()

            @plsc.parallel_loop(0, W, unroll=4)
            def _edge(e):
                acc = zvec
                xjc = []
                for k in range(D // 16):
                    xj_c = xj_v[e, pl.ds(k * 16, 16)]
                    xi_c = xi_v[e, pl.ds(k * 16, 16)]
                    z = xi_c + xj_c
                    lr = jnp.maximum(z, z * NEG)
                    acc = acc + lr * att_v[h, pl.ds(k * 16, 16)]
                    xjc.append(xj_c)
                pv = jnp.exp(jnp.full((16,), jnp.sum(acc)))
                for k in range(D // 16):
                    xj_v[e, pl.ds(k * 16, 16)] = xjc[k] * pv
                dchunk = didx_e[pl.ds(e, 16)]
                plsc.addupdate_scatter(den_t, [dchunk], pv, mask=lane0b)

            pltpu.sync_copy(xj_v, out_sp.at[didx_s], add=True)
            return carry

        lax.fori_loop(0, NWIN, _win, 0)
        plsc.subcore_barrier()

        hb = (c * H + h) * N
        obase = hb + s * DRN

        def _drain(k, carry):
            pltpu.sync_copy(out_sp.at[pl.ds(s * DRN + k * 48, 48)],
                            xi_v.at[pl.ds(0, 48)])
            pltpu.sync_copy(xi_v.at[pl.ds(0, 48)],
                            outp.at[pl.ds(obase + k * 48, 48)])
            return carry

        lax.fori_loop(0, DRN // 48, _drain, 0)

        @pl.when(s == 15)
        def _tail():
            pltpu.sync_copy(out_sp.at[pl.ds(16 * DRN, DTL)],
                            xi_v.at[pl.ds(0, DTL)])
            pltpu.sync_copy(xi_v.at[pl.ds(0, DTL)],
                            outp.at[pl.ds(hb + 16 * DRN, DTL)])

        dbase = ((c * H + h) * 16 + s) * NPAD
        pltpu.sync_copy(den_t, denp.at[pl.ds(dbase, NPAD)])

        plsc.subcore_barrier()


def _comb_body(o_ref, d_ref, b_ref, out_ref):
    o = o_ref[...]
    dn = d_ref[...]
    sel = jnp.where(
        lax.broadcasted_iota(jnp.int32, (128, 2 * H), 0) // 16
        == lax.broadcasted_iota(jnp.int32, (128, 2 * H), 1), 1.0, 0.0)
    dsum = jnp.dot(dn, sel, preferred_element_type=jnp.float32)
    o2 = o[0:H] + o[H:2 * H]
    acc = jnp.zeros_like(o2[0])
    for h in range(H):
        den_h = dsum[:, h:h + 1] + dsum[:, H + h:H + h + 1] + 1e-16
        acc = acc + o2[h] / den_h
    out_ref[...] = acc * (1.0 / H) + b_ref[...]


def _combine(outp, denp, bias):
    den_t = denp.reshape(2 * H * 16, NPAD).T
    return pl.pallas_call(
        _comb_body,
        grid=(N // BS,),
        in_specs=[
            pl.BlockSpec((2 * H, BS, D), lambda i: (0, i, 0)),
            pl.BlockSpec((BS, 128), lambda i: (i, 0)),
            pl.BlockSpec((1, D), lambda i: (0, 0)),
        ],
        out_specs=pl.BlockSpec((BS, D), lambda i: (i, 0)),
        out_shape=jax.ShapeDtypeStruct((N, D), jnp.float32),
    )(outp.reshape(2 * H, N, D), den_t, bias.reshape(1, D))


def kernel(x, edge_index, W_l, W_r, att, bias):
    xlf, xrf = _proj(x, W_l, W_r)

    loop = jnp.arange(N, dtype=jnp.int32)
    padi = jnp.arange(EPAD - ETOT, dtype=jnp.int32)
    srcp = jnp.concatenate([edge_index[0], loop, padi % N])
    dstp = jnp.concatenate([edge_index[1], loop, N + padi % (NPAD - N)])

    scfn = pl.kernel(
        _sc_body,
        out_type=[
            jax.ShapeDtypeStruct((2 * H * N, D), jnp.float32),
            jax.ShapeDtypeStruct((2 * H * 16 * NPAD,), jnp.float32),
        ],
        mesh=plsc.VectorSubcoreMesh(core_axis_name="c", subcore_axis_name="s"),
        compiler_params=pltpu.CompilerParams(needs_layout_passes=False),
        scratch_types=[
            pltpu.VMEM((H, D), jnp.float32),
            pltpu.VMEM((W,), jnp.int32),
            pltpu.VMEM((W,), jnp.int32),
            pltpu.VMEM((W + 16,), jnp.int32),
            pltpu.VMEM((W,), jnp.int32),
            pltpu.VMEM((W, D), jnp.float32),
            pltpu.VMEM((W, D), jnp.float32),
            pltpu.VMEM((NPAD,), jnp.float32),
            pltpu.VMEM((ZCH, D), jnp.float32),
            pltpu.VMEM_SHARED((NPAD, D), jnp.float32),
            pltpu.SemaphoreType.DMA,
            pltpu.SemaphoreType.DMA,
        ],
    )
    outp, denp = scfn(xlf, xrf, srcp, dstp, att)
    return _combine(outp, denp, bias)
